# Initial kernel scaffold; baseline (speedup 1.0000x reference)
import jax
import jax.numpy as jnp
from jax.experimental import pallas as pl


def kernel(inputs, weight):
    raise NotImplementedError("write your pallas kernel here")


# speedup vs baseline: 2.7985x; 2.7985x over previous
import functools

import jax
import jax.numpy as jnp
from jax.experimental import pallas as pl
from jax.experimental.pallas import tpu as pltpu

N_ROWS = 18432
N_STATES = 1024
Z_DIM = 64
BLOCK = 512
N_BLOCKS = N_ROWS // BLOCK
COMMITMENT_COST = 0.25


def _vq_kernel(x_ref, w_ref, loss_ref, q_ref, perp_ref, enc_ref,
               counts_ref, sse_ref):
    i = pl.program_id(0)
    x = x_ref[...]
    w = w_ref[...]

    rn = jnp.sum(x * x, axis=1, keepdims=True)
    wn = jnp.sum(w * w, axis=1).reshape(1, N_STATES)
    mm = jax.lax.dot_general(x, w, (((1,), (1,)), ((), ())),
                             preferred_element_type=jnp.float32)
    d = rn + wn - 2.0 * mm

    m = jnp.min(d, axis=1, keepdims=True)
    ii = jax.lax.broadcasted_iota(jnp.int32, (BLOCK, N_STATES), 1)
    idx = jnp.min(jnp.where(d == m, ii, N_STATES), axis=1, keepdims=True)
    onehot = (ii == idx).astype(jnp.float32)
    enc_ref[...] = onehot

    q = jax.lax.dot_general(onehot, w, (((1,), (0,)), ((), ())),
                            preferred_element_type=jnp.float32)
    dq = q - x
    q_ref[...] = x + dq

    @pl.when(i == 0)
    def _init():
        counts_ref[...] = jnp.zeros_like(counts_ref)
        sse_ref[...] = jnp.zeros_like(sse_ref)

    counts_ref[...] += jnp.sum(onehot, axis=0, keepdims=True)
    sse_ref[...] += jnp.sum(dq * dq, keepdims=True)

    @pl.when(i == N_BLOCKS - 1)
    def _fini():
        sse = sse_ref[0, 0]
        loss_ref[...] = jnp.full((1, 1), (1.0 + COMMITMENT_COST)
                                 * sse / (N_ROWS * Z_DIM))
        avg = counts_ref[...] / N_ROWS
        ent = jnp.sum(avg * jnp.log(avg + 1e-10), keepdims=True)
        perp_ref[...] = jnp.exp(-ent)


@jax.jit
def kernel(inputs, weight):
    loss, quantized_st, perp, encodings = pl.pallas_call(
        _vq_kernel,
        grid=(N_BLOCKS,),
        in_specs=[
            pl.BlockSpec((BLOCK, Z_DIM), lambda i: (i, 0)),
            pl.BlockSpec((N_STATES, Z_DIM), lambda i: (0, 0)),
        ],
        out_specs=[
            pl.BlockSpec((1, 1), lambda i: (0, 0)),
            pl.BlockSpec((BLOCK, Z_DIM), lambda i: (i, 0)),
            pl.BlockSpec((1, 1), lambda i: (0, 0)),
            pl.BlockSpec((BLOCK, N_STATES), lambda i: (i, 0)),
        ],
        out_shape=[
            jax.ShapeDtypeStruct((1, 1), jnp.float32),
            jax.ShapeDtypeStruct((N_ROWS, Z_DIM), jnp.float32),
            jax.ShapeDtypeStruct((1, 1), jnp.float32),
            jax.ShapeDtypeStruct((N_ROWS, N_STATES), jnp.float32),
        ],
        scratch_shapes=[
            pltpu.VMEM((1, N_STATES), jnp.float32),
            pltpu.VMEM((1, 1), jnp.float32),
        ],
    )(inputs, weight)
    return (loss.reshape(()), quantized_st, perp.reshape(()), encodings)


# speedup vs baseline: 3.7450x; 1.3382x over previous
import functools

import jax
import jax.numpy as jnp
from jax.experimental import pallas as pl
from jax.experimental.pallas import tpu as pltpu

N_ROWS = 18432
N_STATES = 1024
Z_DIM = 64
BLOCK = 1024
N_BLOCKS = N_ROWS // BLOCK
COMMITMENT_COST = 0.25


def _vq_kernel(x_ref, w_ref, loss_ref, q_ref, perp_ref, enc_ref,
               wn_ref, iota_ref, counts_ref, sse_ref):
    i = pl.program_id(0)
    x = x_ref[...]
    w = w_ref[...]

    @pl.when(i == 0)
    def _init():
        wn_ref[...] = jnp.sum(w * w, axis=1).reshape(1, N_STATES)
        iota_ref[...] = jax.lax.broadcasted_iota(
            jnp.int32, (1, N_STATES), 1).astype(jnp.float32)
        counts_ref[...] = jnp.zeros_like(counts_ref)
        sse_ref[...] = jnp.zeros_like(sse_ref)

    rn = jnp.sum(x * x, axis=1, keepdims=True)
    wn = wn_ref[...]
    mm2 = jax.lax.dot_general(x + x, w, (((1,), (1,)), ((), ())),
                              preferred_element_type=jnp.float32)
    d = rn + wn - mm2

    m = jnp.min(d, axis=1, keepdims=True)
    ii = iota_ref[...]
    idx = jnp.min(jnp.where(d == m, ii, jnp.float32(N_STATES)),
                  axis=1, keepdims=True)
    onehot = (ii == idx).astype(jnp.float32)
    enc_ref[...] = onehot

    q = jax.lax.dot_general(onehot, w, (((1,), (0,)), ((), ())),
                            preferred_element_type=jnp.float32)
    dq = q - x
    q_ref[...] = x + dq

    ones_row = jnp.ones((1, BLOCK), jnp.float32)
    counts_ref[...] += jax.lax.dot_general(
        ones_row, onehot, (((1,), (0,)), ((), ())),
        preferred_element_type=jnp.float32)
    sse_ref[...] += jnp.sum(dq * dq, keepdims=True)

    @pl.when(i == N_BLOCKS - 1)
    def _fini():
        sse = sse_ref[0, 0]
        loss_ref[...] = jnp.full((1, 1), (1.0 + COMMITMENT_COST)
                                 * sse / (N_ROWS * Z_DIM))
        avg = counts_ref[...] / N_ROWS
        ent = jnp.sum(avg * jnp.log(avg + 1e-10), keepdims=True)
        perp_ref[...] = jnp.exp(-ent)


@jax.jit
def kernel(inputs, weight):
    loss, quantized_st, perp, encodings = pl.pallas_call(
        _vq_kernel,
        grid=(N_BLOCKS,),
        in_specs=[
            pl.BlockSpec((BLOCK, Z_DIM), lambda i: (i, 0)),
            pl.BlockSpec((N_STATES, Z_DIM), lambda i: (0, 0)),
        ],
        out_specs=[
            pl.BlockSpec((1, 1), lambda i: (0, 0)),
            pl.BlockSpec((BLOCK, Z_DIM), lambda i: (i, 0)),
            pl.BlockSpec((1, 1), lambda i: (0, 0)),
            pl.BlockSpec((BLOCK, N_STATES), lambda i: (i, 0)),
        ],
        out_shape=[
            jax.ShapeDtypeStruct((1, 1), jnp.float32),
            jax.ShapeDtypeStruct((N_ROWS, Z_DIM), jnp.float32),
            jax.ShapeDtypeStruct((1, 1), jnp.float32),
            jax.ShapeDtypeStruct((N_ROWS, N_STATES), jnp.float32),
        ],
        scratch_shapes=[
            pltpu.VMEM((1, N_STATES), jnp.float32),
            pltpu.VMEM((1, N_STATES), jnp.float32),
            pltpu.VMEM((1, N_STATES), jnp.float32),
            pltpu.VMEM((1, 1), jnp.float32),
        ],
    )(inputs, weight)
    return (loss.reshape(()), quantized_st, perp.reshape(()), encodings)
